# Initial kernel scaffold; baseline (speedup 1.0000x reference)
import jax
import jax.numpy as jnp
from jax.experimental import pallas as pl


def kernel(voxel_coords, voxel_coords_batch, voxel_x, params):
    raise NotImplementedError("write your pallas kernel here")


# speedup vs baseline: 3.3177x; 3.3177x over previous
import functools

import jax
import jax.numpy as jnp
from jax.experimental import pallas as pl
from jax.experimental.pallas import tpu as pltpu

N4, N3, N2, N1 = 686, 5488, 43904, 351232
N4P, N3P, N2P, N1P = 688, 5504, 44032, 352256
N0D = 2_000_000
N_PTS = 100_000
PB = 1024
N_PTS_PAD = 100_352

def _relu(x):
    return jnp.maximum(x, 0.0)


def _dot(a, w):
    return jax.lax.dot_general(a, w, (((1,), (0,)), ((), ())),
                               preferred_element_type=jnp.float32)


def _pt0_body(f0p, s1, s2, t1, t2, z0_o, zw40_o, r1_o, r1pt2_o):
    x = _relu(_dot(f0p[:, :4], s1[...]))
    z0 = _relu(_dot(x, s2[...]))
    w = f0p[:, 4:5]
    z0_o[...] = z0
    zw40_o[...] = jnp.concatenate(
        [z0 * w, w, jnp.zeros((z0.shape[0], 7), jnp.float32)], 1)
    r1 = _relu(_dot(z0, t1[...]))
    r1_o[...] = r1
    r1pt2_o[...] = _dot(r1, t2[...])


def _tc_pt0(f0p_w, p):
    grid = N_PTS_PAD // PB
    return pl.pallas_call(
        lambda f, s1, s2, t1, t2, a, b, c, d: _pt0_body(
            f[...], s1, s2, t1, t2, a, b, c, d),
        grid=(grid,),
        in_specs=[
            pl.BlockSpec((PB, 8), lambda i: (i, 0)),
            pl.BlockSpec((4, 32), lambda i: (0, 0)),
            pl.BlockSpec((32, 32), lambda i: (0, 0)),
            pl.BlockSpec((32, 256), lambda i: (0, 0)),
            pl.BlockSpec((256, 128), lambda i: (0, 0)),
        ],
        out_specs=[
            pl.BlockSpec((PB, 32), lambda i: (i, 0)),
            pl.BlockSpec((PB, 40), lambda i: (i, 0)),
            pl.BlockSpec((PB, 256), lambda i: (i, 0)),
            pl.BlockSpec((PB, 128), lambda i: (i, 0)),
        ],
        out_shape=[
            jax.ShapeDtypeStruct((N_PTS_PAD, 32), jnp.float32),
            jax.ShapeDtypeStruct((N_PTS_PAD, 40), jnp.float32),
            jax.ShapeDtypeStruct((N_PTS_PAD, 256), jnp.float32),
            jax.ShapeDtypeStruct((N_PTS_PAD, 128), jnp.float32),
        ],
    )(f0p_w, p['stem1'], p['stem2'], p['pt1'], p['pt2'])


def _down_body(tin, wa, wb, x_o, nxt_o, cin, cout):
    R = tin.shape[0]
    s = jnp.zeros((R, cout), jnp.float32)
    m = jnp.zeros((R, 1), jnp.float32)
    for j in range(8):
        tj = tin[:, j, :]
        mj = tj[:, cin:cin + 1]
        aj = tj[:, :cin] / jnp.maximum(mj, 1.0)
        xj = _relu(_dot(_relu(_dot(aj, wa[...])), wb[...]))
        x_o[:, j, :] = xj
        s = s + xj
        m = m + (mj > 0.5).astype(jnp.float32)
    nxt_o[:, :cout] = s
    nxt_o[:, cout:cout + 1] = m


def _tc_down(tbl, wa, wb, nrows_p, rblk, cin, cout):
    grid = nrows_p // rblk
    fin, fout = wa.shape[0], wa.shape[1]
    tw = tbl.shape[2]
    return pl.pallas_call(
        lambda t, a, b, xo, no: _down_body(t[...], a, b, xo, no, cin, cout),
        grid=(grid,),
        in_specs=[
            pl.BlockSpec((rblk, 8, tw), lambda i: (i, 0, 0)),
            pl.BlockSpec((fin, fout), lambda i: (0, 0)),
            pl.BlockSpec((fout, cout), lambda i: (0, 0)),
        ],
        out_specs=[
            pl.BlockSpec((rblk, 8, cout), lambda i: (i, 0, 0)),
            pl.BlockSpec((rblk, cout + 1), lambda i: (i, 0)),
        ],
        out_shape=[
            jax.ShapeDtypeStruct((nrows_p, 8, cout), jnp.float32),
            jax.ShapeDtypeStruct((nrows_p, cout + 1), jnp.float32),
        ],
    )(tbl, wa, wb)


def _lvl4_body(x4in, s4a, s4b, t2, x4_o, x4pt2_o):
    a = x4in[:, :128] / jnp.maximum(x4in[:, 128:129], 1.0)
    x4 = _relu(_dot(_relu(_dot(a, s4a[...])), s4b[...]))
    x4_o[...] = x4
    x4pt2_o[...] = _dot(x4, t2[...])


def _tc_lvl4(x4in, p):
    return pl.pallas_call(
        _lvl4_body,
        in_specs=[pl.BlockSpec((N4P, 129), lambda: (0, 0)),
                  pl.BlockSpec((128, 256), lambda: (0, 0)),
                  pl.BlockSpec((256, 256), lambda: (0, 0)),
                  pl.BlockSpec((256, 128), lambda: (0, 0))],
        out_specs=[pl.BlockSpec((N4P, 256), lambda: (0, 0)),
                   pl.BlockSpec((N4P, 128), lambda: (0, 0))],
        out_shape=[jax.ShapeDtypeStruct((N4P, 256), jnp.float32),
                   jax.ShapeDtypeStruct((N4P, 128), jnp.float32)],
    )(x4in, p['s4a'], p['s4b'], p['pt2'])


def _up4_body(s4, x4, x3, u1a, u1bt, u1bb, u2a, y2a_o):
    y1in = x4[...] + s4[:, :256] / jnp.maximum(s4[:, 256:257], 1.0)
    y1v = _relu(_dot(y1in, u1a[...]))
    for j in range(8):
        y1j = _relu(_dot(y1v, u1bt[...]) + _dot(x3[:, j, :], u1bb[...]))
        y2a_o[:, j, :] = _relu(_dot(y1j, u2a[...]))


def _tc_up4(s4, x4, x3, p):
    return pl.pallas_call(
        _up4_body,
        in_specs=[pl.BlockSpec((N4P, 272), lambda: (0, 0)),
                  pl.BlockSpec((N4P, 256), lambda: (0, 0)),
                  pl.BlockSpec((N4P, 8, 128), lambda: (0, 0, 0)),
                  pl.BlockSpec((256, 256), lambda: (0, 0)),
                  pl.BlockSpec((256, 256), lambda: (0, 0)),
                  pl.BlockSpec((128, 256), lambda: (0, 0)),
                  pl.BlockSpec((256, 128), lambda: (0, 0))],
        out_specs=pl.BlockSpec((N4P, 8, 128), lambda: (0, 0, 0)),
        out_shape=jax.ShapeDtypeStruct((N4P, 8, 128), jnp.float32),
    )(s4, x4, x3, p['u1a'], p['u1b'][:256], p['u1b'][256:], p['u2a'])


def _up3_body(y2a, x2, u2bt, u2bb, y2_o):
    t = _dot(y2a[...], u2bt[...])
    for j in range(8):
        y2_o[:, j, :] = _relu(t + _dot(x2[:, j, :], u2bb[...]))


def _tc_up3(y2a, x2, p, rblk=64):
    grid = N3P // rblk
    return pl.pallas_call(
        _up3_body,
        grid=(grid,),
        in_specs=[pl.BlockSpec((rblk, 128), lambda i: (i, 0)),
                  pl.BlockSpec((rblk, 8, 64), lambda i: (i, 0, 0)),
                  pl.BlockSpec((128, 128), lambda i: (0, 0)),
                  pl.BlockSpec((64, 128), lambda i: (0, 0))],
        out_specs=pl.BlockSpec((rblk, 8, 128), lambda i: (i, 0, 0)),
        out_shape=jax.ShapeDtypeStruct((N3P, 8, 128), jnp.float32),
    )(y2a, x2, p['u2b'][:128], p['u2b'][128:])


def _y3_body(y2, s2u, u3a, y3a_o):
    y3v = y2[...] + s2u[:, :128] / jnp.maximum(s2u[:, 128:129], 1.0)
    y3a_o[...] = _relu(_dot(y3v, u3a[...]))


def _tc_y3(y2, s2u, p, rblk=512):
    grid = N2P // rblk
    return pl.pallas_call(
        _y3_body,
        grid=(grid,),
        in_specs=[pl.BlockSpec((rblk, 128), lambda i: (i, 0)),
                  pl.BlockSpec((rblk, 136), lambda i: (i, 0)),
                  pl.BlockSpec((128, 96), lambda i: (0, 0))],
        out_specs=pl.BlockSpec((rblk, 96), lambda i: (i, 0)),
        out_shape=jax.ShapeDtypeStruct((N2P, 96), jnp.float32),
    )(y2, s2u, p['u3a'])


def _final_body(gy3a, gx1, z0, y2p, r2p, u3bt, u3bb, u4a, u4bt, u4bb,
                pt3, clsw, clsb, out_o):
    z2 = y2p[...] + r2p[...]
    y3p = _relu(_dot(gy3a[...], u3bt[...]) + _dot(gx1[...], u3bb[...]))
    y4a = _relu(_dot(y3p, u4a[...]))
    y4p = _relu(_dot(y4a, u4bt[...]) + _dot(z0[...], u4bb[...]))
    z3 = y4p + _relu(_dot(z2, pt3[...]))
    out_o[...] = _dot(z3, clsw[...]) + clsb[...]


def _tc_final(gy3a, gx1, z0, y2p, r2p, p):
    grid = N_PTS_PAD // PB
    return pl.pallas_call(
        _final_body,
        grid=(grid,),
        in_specs=[pl.BlockSpec((PB, 96), lambda i: (i, 0)),
                  pl.BlockSpec((PB, 32), lambda i: (i, 0)),
                  pl.BlockSpec((PB, 32), lambda i: (i, 0)),
                  pl.BlockSpec((PB, 128), lambda i: (i, 0)),
                  pl.BlockSpec((PB, 128), lambda i: (i, 0)),
                  pl.BlockSpec((96, 96), lambda i: (0, 0)),
                  pl.BlockSpec((32, 96), lambda i: (0, 0)),
                  pl.BlockSpec((96, 96), lambda i: (0, 0)),
                  pl.BlockSpec((96, 96), lambda i: (0, 0)),
                  pl.BlockSpec((32, 96), lambda i: (0, 0)),
                  pl.BlockSpec((128, 96), lambda i: (0, 0)),
                  pl.BlockSpec((96, 20), lambda i: (0, 0)),
                  pl.BlockSpec((1, 20), lambda i: (0, 0))],
        out_specs=pl.BlockSpec((PB, 20), lambda i: (i, 0)),
        out_shape=jax.ShapeDtypeStruct((N_PTS_PAD, 20), jnp.float32),
    )(gy3a, gx1, z0, y2p, r2p, p['u3b'][:96], p['u3b'][96:], p['u4a'],
      p['u4b'][:96], p['u4b'][96:], p['pt3'], p['cls_w'],
      p['cls_b'].reshape(1, 20))


def _r2_body(g4p, r1pt2, r2_o):
    r2_o[...] = _relu(g4p[...] + r1pt2[...])


def _tc_r2(g4p, r1pt2):
    grid = N_PTS_PAD // PB
    return pl.pallas_call(
        _r2_body,
        grid=(grid,),
        in_specs=[pl.BlockSpec((PB, 128), lambda i: (i, 0)),
                  pl.BlockSpec((PB, 128), lambda i: (i, 0))],
        out_specs=pl.BlockSpec((PB, 128), lambda i: (i, 0)),
        out_shape=jax.ShapeDtypeStruct((N_PTS_PAD, 128), jnp.float32),
    )(g4p, r1pt2)


def _seg_sum(x, idx, num):
    return jax.ops.segment_sum(x, idx, num_segments=num)


def _pad_rows(x, n):
    return jnp.pad(x, ((0, n - x.shape[0]),) + ((0, 0),) * (x.ndim - 1))


def kernel(voxel_coords, voxel_coords_batch, voxel_x, params):
    p = params
    c = voxel_coords.astype(jnp.int32)
    b = voxel_coords_batch.astype(jnp.int32)
    g = c // 16
    k4 = ((g[:, 0] * 7 + g[:, 1]) * 7 + g[:, 2]) * 2 + b
    k3 = k4 * 8 + ((c[:, 0] // 8) % 2) * 4 + ((c[:, 1] // 8) % 2) * 2 + ((c[:, 2] // 8) % 2)
    k2 = k3 * 8 + ((c[:, 0] // 4) % 2) * 4 + ((c[:, 1] // 4) % 2) * 2 + ((c[:, 2] // 4) % 2)
    k1 = k2 * 8 + ((c[:, 0] // 2) % 2) * 4 + ((c[:, 1] // 2) % 2) * 2 + ((c[:, 2] // 2) % 2)
    k0 = (c[:, 0] * 100 + c[:, 1]) * 100 + c[:, 2]
    k0 = k0 * 2 + b

    ones = jnp.ones((N_PTS, 1), jnp.float32)
    t0 = _seg_sum(jnp.concatenate([voxel_x, ones], 1), k0, N0D)
    t0p = t0[k0]
    cnt0 = jnp.maximum(t0p[:, 4:5], 1.0)
    f0p_w = jnp.concatenate(
        [t0p[:, :4] / cnt0, 1.0 / cnt0,
         jnp.zeros((N_PTS, 3), jnp.float32)], 1)
    f0p_w = _pad_rows(f0p_w, N_PTS_PAD)

    z0, zw40, r1, r1pt2 = _tc_pt0(f0p_w, p)

    t1 = _seg_sum(zw40[:N_PTS], k1, N1)
    t1 = _pad_rows(t1, N1P).reshape(N2P, 8, 40)

    x1, x2in = _tc_down(t1, p['s1a'], p['s1b'], N2P, 512, 32, 32)
    x2in = x2in.reshape(N3P, 8, 33)
    x2, x3in = _tc_down(x2in, p['s2a'], p['s2b'], N3P, 64, 32, 64)
    x3in = x3in.reshape(N4P, 8, 65)
    x3, x4in = _tc_down(x3in, p['s3a'], p['s3b'], N4P, N4P, 64, 128)
    x4, x4pt2 = _tc_lvl4(x4in, p)

    s4 = _seg_sum(jnp.concatenate([r1[:N_PTS], ones], 1), k4, N4)
    s4 = jnp.pad(s4, ((0, N4P - N4), (0, 272 - 257)))
    g4p = _pad_rows(x4pt2[k4], N_PTS_PAD)

    r2p = _tc_r2(g4p, r1pt2)

    s2u = _seg_sum(jnp.concatenate([r2p[:N_PTS], ones], 1), k2, N2)
    s2u = jnp.pad(s2u, ((0, N2P - N2), (0, 136 - 129)))

    y2a = _tc_up4(s4, x4, x3, p)
    y2 = _tc_up3(y2a.reshape(N3P, 128), x2.reshape(N3P, 8, 64), p)
    y2 = y2.reshape(N2P, 128)
    y3a = _tc_y3(y2, s2u, p)

    gy3a = _pad_rows(y3a[k2], N_PTS_PAD)
    gx1 = _pad_rows(x1.reshape(N1P, 32)[k1], N_PTS_PAD)
    y2p = _pad_rows(y2[k2], N_PTS_PAD)

    out = _tc_final(gy3a, gx1, z0, y2p, r2p, p)
    return out[:N_PTS]


# speedup vs baseline: 3.9606x; 1.1938x over previous
import functools

import jax
import jax.numpy as jnp
from jax.experimental import pallas as pl
from jax.experimental.pallas import tpu as pltpu

N4, N3, N2, N1 = 686, 5488, 43904, 351232
N4P, N3P, N2P, N1P = 688, 5504, 44032, 352256
N0D = 2_000_000
N_PTS = 100_000
PB = 1024
N_PTS_PAD = 100_352

def _relu(x):
    return jnp.maximum(x, 0.0)


def _dot(a, w):
    return jax.lax.dot_general(a, w, (((1,), (0,)), ((), ())),
                               preferred_element_type=jnp.float32)


def _pt0_body(f0p, s1, s2, t1, t2, z0_o, zw40_o, r1_o, r1pt2_o):
    x = _relu(_dot(f0p[:, :4], s1[...]))
    z0 = _relu(_dot(x, s2[...]))
    w = f0p[:, 4:5]
    z0_o[...] = z0
    zw40_o[...] = jnp.concatenate(
        [z0 * w, w, jnp.zeros((z0.shape[0], 7), jnp.float32)], 1)
    r1 = _relu(_dot(z0, t1[...]))
    r1_o[...] = r1
    r1pt2_o[...] = _dot(r1, t2[...])


def _tc_pt0(f0p_w, p):
    grid = N_PTS_PAD // PB
    return pl.pallas_call(
        lambda f, s1, s2, t1, t2, a, b, c, d: _pt0_body(
            f[...], s1, s2, t1, t2, a, b, c, d),
        grid=(grid,),
        in_specs=[
            pl.BlockSpec((PB, 8), lambda i: (i, 0)),
            pl.BlockSpec((4, 32), lambda i: (0, 0)),
            pl.BlockSpec((32, 32), lambda i: (0, 0)),
            pl.BlockSpec((32, 256), lambda i: (0, 0)),
            pl.BlockSpec((256, 128), lambda i: (0, 0)),
        ],
        out_specs=[
            pl.BlockSpec((PB, 32), lambda i: (i, 0)),
            pl.BlockSpec((PB, 40), lambda i: (i, 0)),
            pl.BlockSpec((PB, 256), lambda i: (i, 0)),
            pl.BlockSpec((PB, 128), lambda i: (i, 0)),
        ],
        out_shape=[
            jax.ShapeDtypeStruct((N_PTS_PAD, 32), jnp.float32),
            jax.ShapeDtypeStruct((N_PTS_PAD, 40), jnp.float32),
            jax.ShapeDtypeStruct((N_PTS_PAD, 256), jnp.float32),
            jax.ShapeDtypeStruct((N_PTS_PAD, 128), jnp.float32),
        ],
    )(f0p_w, p['stem1'], p['stem2'], p['pt1'], p['pt2'])


def _down_body(tin, wa, wb, x_o, nxt_o, cin, cout):
    R = tin.shape[0]
    s = jnp.zeros((R, cout), jnp.float32)
    m = jnp.zeros((R, 1), jnp.float32)
    for j in range(8):
        tj = tin[:, j, :]
        mj = tj[:, cin:cin + 1]
        aj = tj[:, :cin] / jnp.maximum(mj, 1.0)
        xj = _relu(_dot(_relu(_dot(aj, wa[...])), wb[...]))
        x_o[:, j, :] = xj
        s = s + xj
        m = m + (mj > 0.5).astype(jnp.float32)
    nxt_o[:, :cout] = s
    nxt_o[:, cout:cout + 1] = m


def _tc_down(tbl, wa, wb, nrows_p, rblk, cin, cout):
    grid = nrows_p // rblk
    fin, fout = wa.shape[0], wa.shape[1]
    tw = tbl.shape[2]
    return pl.pallas_call(
        lambda t, a, b, xo, no: _down_body(t[...], a, b, xo, no, cin, cout),
        grid=(grid,),
        in_specs=[
            pl.BlockSpec((rblk, 8, tw), lambda i: (i, 0, 0)),
            pl.BlockSpec((fin, fout), lambda i: (0, 0)),
            pl.BlockSpec((fout, cout), lambda i: (0, 0)),
        ],
        out_specs=[
            pl.BlockSpec((rblk, 8, cout), lambda i: (i, 0, 0)),
            pl.BlockSpec((rblk, cout + 1), lambda i: (i, 0)),
        ],
        out_shape=[
            jax.ShapeDtypeStruct((nrows_p, 8, cout), jnp.float32),
            jax.ShapeDtypeStruct((nrows_p, cout + 1), jnp.float32),
        ],
    )(tbl, wa, wb)


def _lvl4_body(x4in, s4a, s4b, t2, x4_o, x4pt2_o):
    a = x4in[:, :128] / jnp.maximum(x4in[:, 128:129], 1.0)
    x4 = _relu(_dot(_relu(_dot(a, s4a[...])), s4b[...]))
    x4_o[...] = x4
    x4pt2_o[...] = _dot(x4, t2[...])


def _tc_lvl4(x4in, p):
    return pl.pallas_call(
        _lvl4_body,
        in_specs=[pl.BlockSpec((N4P, 129), lambda: (0, 0)),
                  pl.BlockSpec((128, 256), lambda: (0, 0)),
                  pl.BlockSpec((256, 256), lambda: (0, 0)),
                  pl.BlockSpec((256, 128), lambda: (0, 0))],
        out_specs=[pl.BlockSpec((N4P, 256), lambda: (0, 0)),
                   pl.BlockSpec((N4P, 128), lambda: (0, 0))],
        out_shape=[jax.ShapeDtypeStruct((N4P, 256), jnp.float32),
                   jax.ShapeDtypeStruct((N4P, 128), jnp.float32)],
    )(x4in, p['s4a'], p['s4b'], p['pt2'])


def _up4_body(s4, x4, x3, u1a, u1bt, u1bb, u2a, y2a_o):
    y1in = x4[...] + s4[:, :256] / jnp.maximum(s4[:, 256:257], 1.0)
    y1v = _relu(_dot(y1in, u1a[...]))
    for j in range(8):
        y1j = _relu(_dot(y1v, u1bt[...]) + _dot(x3[:, j, :], u1bb[...]))
        y2a_o[:, j, :] = _relu(_dot(y1j, u2a[...]))


def _tc_up4(s4, x4, x3, p):
    return pl.pallas_call(
        _up4_body,
        in_specs=[pl.BlockSpec((N4P, 272), lambda: (0, 0)),
                  pl.BlockSpec((N4P, 256), lambda: (0, 0)),
                  pl.BlockSpec((N4P, 8, 128), lambda: (0, 0, 0)),
                  pl.BlockSpec((256, 256), lambda: (0, 0)),
                  pl.BlockSpec((256, 256), lambda: (0, 0)),
                  pl.BlockSpec((128, 256), lambda: (0, 0)),
                  pl.BlockSpec((256, 128), lambda: (0, 0))],
        out_specs=pl.BlockSpec((N4P, 8, 128), lambda: (0, 0, 0)),
        out_shape=jax.ShapeDtypeStruct((N4P, 8, 128), jnp.float32),
    )(s4, x4, x3, p['u1a'], p['u1b'][:256], p['u1b'][256:], p['u2a'])


def _up3_body(y2a, x2, u2bt, u2bb, y2_o):
    t = _dot(y2a[...], u2bt[...])
    for j in range(8):
        y2_o[:, j, :] = _relu(t + _dot(x2[:, j, :], u2bb[...]))


def _tc_up3(y2a, x2, p, rblk=64):
    grid = N3P // rblk
    return pl.pallas_call(
        _up3_body,
        grid=(grid,),
        in_specs=[pl.BlockSpec((rblk, 128), lambda i: (i, 0)),
                  pl.BlockSpec((rblk, 8, 64), lambda i: (i, 0, 0)),
                  pl.BlockSpec((128, 128), lambda i: (0, 0)),
                  pl.BlockSpec((64, 128), lambda i: (0, 0))],
        out_specs=pl.BlockSpec((rblk, 8, 128), lambda i: (i, 0, 0)),
        out_shape=jax.ShapeDtypeStruct((N3P, 8, 128), jnp.float32),
    )(y2a, x2, p['u2b'][:128], p['u2b'][128:])


def _y3_body(y2, s2u, u3a, y3a_o):
    y3v = y2[...] + s2u[:, :128] / jnp.maximum(s2u[:, 128:129], 1.0)
    y3a_o[...] = _relu(_dot(y3v, u3a[...]))


def _tc_y3(y2, s2u, p, rblk=512):
    grid = N2P // rblk
    return pl.pallas_call(
        _y3_body,
        grid=(grid,),
        in_specs=[pl.BlockSpec((rblk, 128), lambda i: (i, 0)),
                  pl.BlockSpec((rblk, 136), lambda i: (i, 0)),
                  pl.BlockSpec((128, 96), lambda i: (0, 0))],
        out_specs=pl.BlockSpec((rblk, 96), lambda i: (i, 0)),
        out_shape=jax.ShapeDtypeStruct((N2P, 96), jnp.float32),
    )(y2, s2u, p['u3a'])


def _final_body(gy3a, gx1, z0, y2p, r2p, u3bt, u3bb, u4a, u4bt, u4bb,
                pt3, clsw, clsb, out_o):
    z2 = y2p[...] + r2p[...]
    y3p = _relu(_dot(gy3a[...], u3bt[...]) + _dot(gx1[...], u3bb[...]))
    y4a = _relu(_dot(y3p, u4a[...]))
    y4p = _relu(_dot(y4a, u4bt[...]) + _dot(z0[...], u4bb[...]))
    z3 = y4p + _relu(_dot(z2, pt3[...]))
    out_o[...] = _dot(z3, clsw[...]) + clsb[...]


def _tc_final(gy3a, gx1, z0, y2p, r2p, p):
    grid = N_PTS_PAD // PB
    return pl.pallas_call(
        _final_body,
        grid=(grid,),
        in_specs=[pl.BlockSpec((PB, 96), lambda i: (i, 0)),
                  pl.BlockSpec((PB, 32), lambda i: (i, 0)),
                  pl.BlockSpec((PB, 32), lambda i: (i, 0)),
                  pl.BlockSpec((PB, 128), lambda i: (i, 0)),
                  pl.BlockSpec((PB, 128), lambda i: (i, 0)),
                  pl.BlockSpec((96, 96), lambda i: (0, 0)),
                  pl.BlockSpec((32, 96), lambda i: (0, 0)),
                  pl.BlockSpec((96, 96), lambda i: (0, 0)),
                  pl.BlockSpec((96, 96), lambda i: (0, 0)),
                  pl.BlockSpec((32, 96), lambda i: (0, 0)),
                  pl.BlockSpec((128, 96), lambda i: (0, 0)),
                  pl.BlockSpec((96, 20), lambda i: (0, 0)),
                  pl.BlockSpec((1, 20), lambda i: (0, 0))],
        out_specs=pl.BlockSpec((PB, 20), lambda i: (i, 0)),
        out_shape=jax.ShapeDtypeStruct((N_PTS_PAD, 20), jnp.float32),
    )(gy3a, gx1, z0, y2p, r2p, p['u3b'][:96], p['u3b'][96:], p['u4a'],
      p['u4b'][:96], p['u4b'][96:], p['pt3'], p['cls_w'],
      p['cls_b'].reshape(1, 20))


def _r2_body(k4r, x4pt2, r1, r1pt2, r2_o, s4_o):
    i = pl.program_id(0)
    kb = k4r[0]
    mask = (kb == jax.lax.broadcasted_iota(jnp.int32, (PB, 768), 1))
    maskf = mask.astype(jnp.float32)
    g4 = _dot(maskf, x4pt2[...])
    r2_o[...] = _relu(g4 + r1pt2[...])
    r1aug = jnp.concatenate(
        [r1[...], jnp.ones((PB, 1), jnp.float32),
         jnp.zeros((PB, 15), jnp.float32)], 1)
    part = jax.lax.dot_general(maskf, r1aug, (((0,), (0,)), ((), ())),
                               preferred_element_type=jnp.float32)

    @pl.when(i == 0)
    def _():
        s4_o[...] = part

    @pl.when(i != 0)
    def _():
        s4_o[...] += part


def _tc_r2(k4r, x4pt2_768, r1, r1pt2):
    grid = N_PTS_PAD // PB
    return pl.pallas_call(
        _r2_body,
        grid=(grid,),
        in_specs=[pl.BlockSpec((1, PB, 1), lambda i: (i, 0, 0)),
                  pl.BlockSpec((768, 128), lambda i: (0, 0)),
                  pl.BlockSpec((PB, 256), lambda i: (i, 0)),
                  pl.BlockSpec((PB, 128), lambda i: (i, 0))],
        out_specs=[pl.BlockSpec((PB, 128), lambda i: (i, 0)),
                   pl.BlockSpec((768, 272), lambda i: (0, 0))],
        out_shape=[jax.ShapeDtypeStruct((N_PTS_PAD, 128), jnp.float32),
                   jax.ShapeDtypeStruct((768, 272), jnp.float32)],
    )(k4r, x4pt2_768, r1, r1pt2)


def _seg_sum(x, idx, num):
    return jax.ops.segment_sum(x, idx, num_segments=num)


def _pad_rows(x, n):
    return jnp.pad(x, ((0, n - x.shape[0]),) + ((0, 0),) * (x.ndim - 1))


def kernel(voxel_coords, voxel_coords_batch, voxel_x, params):
    p = params
    c = voxel_coords.astype(jnp.int32)
    b = voxel_coords_batch.astype(jnp.int32)
    g = c // 16
    k4 = ((g[:, 0] * 7 + g[:, 1]) * 7 + g[:, 2]) * 2 + b
    k3 = k4 * 8 + ((c[:, 0] // 8) % 2) * 4 + ((c[:, 1] // 8) % 2) * 2 + ((c[:, 2] // 8) % 2)
    k2 = k3 * 8 + ((c[:, 0] // 4) % 2) * 4 + ((c[:, 1] // 4) % 2) * 2 + ((c[:, 2] // 4) % 2)
    k1 = k2 * 8 + ((c[:, 0] // 2) % 2) * 4 + ((c[:, 1] // 2) % 2) * 2 + ((c[:, 2] // 2) % 2)
    k0 = (c[:, 0] * 100 + c[:, 1]) * 100 + c[:, 2]
    k0 = k0 * 2 + b

    ones = jnp.ones((N_PTS, 1), jnp.float32)
    t0 = _seg_sum(jnp.concatenate([voxel_x, ones], 1), k0, N0D)
    t0p = t0[k0]
    cnt0 = jnp.maximum(t0p[:, 4:5], 1.0)
    f0p_w = jnp.concatenate(
        [t0p[:, :4] / cnt0, 1.0 / cnt0,
         jnp.zeros((N_PTS, 3), jnp.float32)], 1)
    f0p_w = _pad_rows(f0p_w, N_PTS_PAD)

    z0, zw40, r1, r1pt2 = _tc_pt0(f0p_w, p)

    t1 = _seg_sum(zw40[:N_PTS], k1, N1)
    t1 = _pad_rows(t1, N1P).reshape(N2P, 8, 40)

    x1, x2in = _tc_down(t1, p['s1a'], p['s1b'], N2P, 512, 32, 32)
    x2in = x2in.reshape(N3P, 8, 33)
    x2, x3in = _tc_down(x2in, p['s2a'], p['s2b'], N3P, 64, 32, 64)
    x3in = x3in.reshape(N4P, 8, 65)
    x3, x4in = _tc_down(x3in, p['s3a'], p['s3b'], N4P, N4P, 64, 128)
    x4, x4pt2 = _tc_lvl4(x4in, p)

    k4r = jnp.pad(k4, (0, N_PTS_PAD - N_PTS), constant_values=N4P) \
             .reshape(N_PTS_PAD // PB, PB, 1)
    x4pt2_768 = _pad_rows(x4pt2, 768)
    r2p, s4f = _tc_r2(k4r, x4pt2_768, r1, r1pt2)
    s4 = s4f[:N4P]

    s2u = _seg_sum(jnp.concatenate([r2p[:N_PTS], ones], 1), k2, N2)
    s2u = jnp.pad(s2u, ((0, N2P - N2), (0, 136 - 129)))

    y2a = _tc_up4(s4, x4, x3, p)
    y2 = _tc_up3(y2a.reshape(N3P, 128), x2.reshape(N3P, 8, 64), p)
    y2 = y2.reshape(N2P, 128)
    y3a = _tc_y3(y2, s2u, p)

    gy3a = _pad_rows(y3a[k2], N_PTS_PAD)
    gx1 = _pad_rows(x1.reshape(N1P, 32)[k1], N_PTS_PAD)
    y2p = _pad_rows(y2[k2], N_PTS_PAD)

    out = _tc_final(gy3a, gx1, z0, y2p, r2p, p)
    return out[:N_PTS]


# speedup vs baseline: 4.5187x; 1.1409x over previous
import functools

import jax
import jax.numpy as jnp
from jax import lax
from jax.experimental import pallas as pl
from jax.experimental.pallas import tpu as pltpu
from jax.experimental.pallas import tpu_sc as plsc

N4, N3, N2, N1 = 686, 5488, 43904, 351232
N4P, N3P, N2P, N1P = 688, 5504, 44032, 352256
N0D = 2_000_000
N_PTS = 100_000
PB = 1024
N_PTS_PAD = 100_352

def _relu(x):
    return jnp.maximum(x, 0.0)


def _dot(a, w):
    return jax.lax.dot_general(a, w, (((1,), (0,)), ((), ())),
                               preferred_element_type=jnp.float32)


def _pt0_body(f0p, s1, s2, t1, t2, z0_o, zw40_o, r1_o, r1pt2_o):
    x = _relu(_dot(f0p[:, :4], s1[...]))
    z0 = _relu(_dot(x, s2[...]))
    w = f0p[:, 4:5]
    z0_o[...] = z0
    zw40_o[...] = jnp.concatenate(
        [z0 * w, w, jnp.zeros((z0.shape[0], 7), jnp.float32)], 1)
    r1 = _relu(_dot(z0, t1[...]))
    r1_o[...] = r1
    r1pt2_o[...] = _dot(r1, t2[...])


def _tc_pt0(f0p_w, p):
    grid = N_PTS_PAD // PB
    return pl.pallas_call(
        lambda f, s1, s2, t1, t2, a, b, c, d: _pt0_body(
            f[...], s1, s2, t1, t2, a, b, c, d),
        grid=(grid,),
        in_specs=[
            pl.BlockSpec((PB, 8), lambda i: (i, 0)),
            pl.BlockSpec((4, 32), lambda i: (0, 0)),
            pl.BlockSpec((32, 32), lambda i: (0, 0)),
            pl.BlockSpec((32, 256), lambda i: (0, 0)),
            pl.BlockSpec((256, 128), lambda i: (0, 0)),
        ],
        out_specs=[
            pl.BlockSpec((PB, 32), lambda i: (i, 0)),
            pl.BlockSpec((PB, 40), lambda i: (i, 0)),
            pl.BlockSpec((PB, 256), lambda i: (i, 0)),
            pl.BlockSpec((PB, 128), lambda i: (i, 0)),
        ],
        out_shape=[
            jax.ShapeDtypeStruct((N_PTS_PAD, 32), jnp.float32),
            jax.ShapeDtypeStruct((N_PTS_PAD, 40), jnp.float32),
            jax.ShapeDtypeStruct((N_PTS_PAD, 256), jnp.float32),
            jax.ShapeDtypeStruct((N_PTS_PAD, 128), jnp.float32),
        ],
    )(f0p_w, p['stem1'], p['stem2'], p['pt1'], p['pt2'])


def _down_body(tin, wa, wb, x_o, nxt_o, cin, cout, xw):
    R = tin.shape[0]
    s = jnp.zeros((R, cout), jnp.float32)
    m = jnp.zeros((R, 1), jnp.float32)
    for j in range(8):
        tj = tin[:, j, :]
        mj = tj[:, cin:cin + 1]
        aj = tj[:, :cin] / jnp.maximum(mj, 1.0)
        xj = _relu(_dot(_relu(_dot(aj, wa[...])), wb[...]))
        if xw > cout:
            xj = jnp.concatenate(
                [xj, jnp.zeros((R, xw - cout), jnp.float32)], 1)
        x_o[:, j, :] = xj
        s = s + xj[:, :cout]
        m = m + (mj > 0.5).astype(jnp.float32)
    nxt_o[:, :cout] = s
    nxt_o[:, cout:cout + 1] = m


def _tc_down(tbl, wa, wb, nrows_p, rblk, cin, cout, xw=None):
    if xw is None:
        xw = cout
    grid = nrows_p // rblk
    fin, fout = wa.shape[0], wa.shape[1]
    tw = tbl.shape[2]
    return pl.pallas_call(
        lambda t, a, b, xo, no: _down_body(t[...], a, b, xo, no, cin, cout, xw),
        grid=(grid,),
        in_specs=[
            pl.BlockSpec((rblk, 8, tw), lambda i: (i, 0, 0)),
            pl.BlockSpec((fin, fout), lambda i: (0, 0)),
            pl.BlockSpec((fout, cout), lambda i: (0, 0)),
        ],
        out_specs=[
            pl.BlockSpec((rblk, 8, xw), lambda i: (i, 0, 0)),
            pl.BlockSpec((rblk, cout + 1), lambda i: (i, 0)),
        ],
        out_shape=[
            jax.ShapeDtypeStruct((nrows_p, 8, xw), jnp.float32),
            jax.ShapeDtypeStruct((nrows_p, cout + 1), jnp.float32),
        ],
    )(tbl, wa, wb)


def _lvl4_body(x4in, s4a, s4b, t2, x4_o, x4pt2_o):
    a = x4in[:, :128] / jnp.maximum(x4in[:, 128:129], 1.0)
    x4 = _relu(_dot(_relu(_dot(a, s4a[...])), s4b[...]))
    x4_o[...] = x4
    x4pt2_o[...] = _dot(x4, t2[...])


def _tc_lvl4(x4in, p):
    return pl.pallas_call(
        _lvl4_body,
        in_specs=[pl.BlockSpec((N4P, 129), lambda: (0, 0)),
                  pl.BlockSpec((128, 256), lambda: (0, 0)),
                  pl.BlockSpec((256, 256), lambda: (0, 0)),
                  pl.BlockSpec((256, 128), lambda: (0, 0))],
        out_specs=[pl.BlockSpec((N4P, 256), lambda: (0, 0)),
                   pl.BlockSpec((N4P, 128), lambda: (0, 0))],
        out_shape=[jax.ShapeDtypeStruct((N4P, 256), jnp.float32),
                   jax.ShapeDtypeStruct((N4P, 128), jnp.float32)],
    )(x4in, p['s4a'], p['s4b'], p['pt2'])


def _up4_body(s4, x4, x3, u1a, u1bt, u1bb, u2a, y2a_o):
    y1in = x4[...] + s4[:, :256] / jnp.maximum(s4[:, 256:257], 1.0)
    y1v = _relu(_dot(y1in, u1a[...]))
    for j in range(8):
        y1j = _relu(_dot(y1v, u1bt[...]) + _dot(x3[:, j, :], u1bb[...]))
        y2a_o[:, j, :] = _relu(_dot(y1j, u2a[...]))


def _tc_up4(s4, x4, x3, p):
    return pl.pallas_call(
        _up4_body,
        in_specs=[pl.BlockSpec((N4P, 272), lambda: (0, 0)),
                  pl.BlockSpec((N4P, 256), lambda: (0, 0)),
                  pl.BlockSpec((N4P, 8, 128), lambda: (0, 0, 0)),
                  pl.BlockSpec((256, 256), lambda: (0, 0)),
                  pl.BlockSpec((256, 256), lambda: (0, 0)),
                  pl.BlockSpec((128, 256), lambda: (0, 0)),
                  pl.BlockSpec((256, 128), lambda: (0, 0))],
        out_specs=pl.BlockSpec((N4P, 8, 128), lambda: (0, 0, 0)),
        out_shape=jax.ShapeDtypeStruct((N4P, 8, 128), jnp.float32),
    )(s4, x4, x3, p['u1a'], p['u1b'][:256], p['u1b'][256:], p['u2a'])


def _up3_body(y2a, x2, u2bt, u2bb, y2_o):
    t = _dot(y2a[...], u2bt[...])
    for j in range(8):
        y2_o[:, j, :] = _relu(t + _dot(x2[:, j, :], u2bb[...]))


def _tc_up3(y2a, x2, p, rblk=64):
    grid = N3P // rblk
    return pl.pallas_call(
        _up3_body,
        grid=(grid,),
        in_specs=[pl.BlockSpec((rblk, 128), lambda i: (i, 0)),
                  pl.BlockSpec((rblk, 8, 64), lambda i: (i, 0, 0)),
                  pl.BlockSpec((128, 128), lambda i: (0, 0)),
                  pl.BlockSpec((64, 128), lambda i: (0, 0))],
        out_specs=pl.BlockSpec((rblk, 8, 128), lambda i: (i, 0, 0)),
        out_shape=jax.ShapeDtypeStruct((N3P, 8, 128), jnp.float32),
    )(y2a, x2, p['u2b'][:128], p['u2b'][128:])


def _y3_body(y2, s2u, u3a, yf_o):
    y3v = y2[...] + s2u[:, :128] / jnp.maximum(s2u[:, 128:129], 1.0)
    yf_o[:, :128] = y2[...]
    yf_o[:, 128:224] = _relu(_dot(y3v, u3a[...]))
    yf_o[:, 224:] = jnp.zeros((y2.shape[0], 32), jnp.float32)


def _tc_y3(y2, s2u, p, rblk=512):
    grid = N2P // rblk
    return pl.pallas_call(
        _y3_body,
        grid=(grid,),
        in_specs=[pl.BlockSpec((rblk, 128), lambda i: (i, 0)),
                  pl.BlockSpec((rblk, 136), lambda i: (i, 0)),
                  pl.BlockSpec((128, 96), lambda i: (0, 0))],
        out_specs=pl.BlockSpec((rblk, 256), lambda i: (i, 0)),
        out_shape=jax.ShapeDtypeStruct((N2P, 256), jnp.float32),
    )(y2, s2u, p['u3a'])


def _final_body(gyf, gx1, z0, r2p, u3bt, u3bb, u4a, u4bt, u4bb,
                pt3, clsw, clsb, out_o):
    gy3a = gyf[:, 128:224]
    z2 = gyf[:, :128] + r2p[...]
    y3p = _relu(_dot(gy3a, u3bt[...]) + _dot(gx1[:, :32], u3bb[...]))
    y4a = _relu(_dot(y3p, u4a[...]))
    y4p = _relu(_dot(y4a, u4bt[...]) + _dot(z0[...], u4bb[...]))
    z3 = y4p + _relu(_dot(z2, pt3[...]))
    out_o[...] = _dot(z3, clsw[...]) + clsb[...]


def _tc_final(gyf, gx1, z0, r2p, p):
    grid = N_PTS_PAD // PB
    return pl.pallas_call(
        _final_body,
        grid=(grid,),
        in_specs=[pl.BlockSpec((PB, 256), lambda i: (i, 0)),
                  pl.BlockSpec((PB, 128), lambda i: (i, 0)),
                  pl.BlockSpec((PB, 32), lambda i: (i, 0)),
                  pl.BlockSpec((PB, 128), lambda i: (i, 0)),
                  pl.BlockSpec((96, 96), lambda i: (0, 0)),
                  pl.BlockSpec((32, 96), lambda i: (0, 0)),
                  pl.BlockSpec((96, 96), lambda i: (0, 0)),
                  pl.BlockSpec((96, 96), lambda i: (0, 0)),
                  pl.BlockSpec((32, 96), lambda i: (0, 0)),
                  pl.BlockSpec((128, 96), lambda i: (0, 0)),
                  pl.BlockSpec((96, 20), lambda i: (0, 0)),
                  pl.BlockSpec((1, 20), lambda i: (0, 0))],
        out_specs=pl.BlockSpec((PB, 20), lambda i: (i, 0)),
        out_shape=jax.ShapeDtypeStruct((N_PTS_PAD, 20), jnp.float32),
    )(gyf, gx1, z0, r2p, p['u3b'][:96], p['u3b'][96:], p['u4a'],
      p['u4b'][:96], p['u4b'][96:], p['pt3'], p['cls_w'],
      p['cls_b'].reshape(1, 20))


def _r2_body(k4r, x4pt2, r1, r1pt2, r2_o, s4_o):
    i = pl.program_id(0)
    kb = k4r[0]
    mask = (kb == jax.lax.broadcasted_iota(jnp.int32, (PB, 768), 1))
    maskf = mask.astype(jnp.float32)
    g4 = _dot(maskf, x4pt2[...])
    r2_o[...] = _relu(g4 + r1pt2[...])
    r1aug = jnp.concatenate(
        [r1[...], jnp.ones((PB, 1), jnp.float32),
         jnp.zeros((PB, 15), jnp.float32)], 1)
    part = jax.lax.dot_general(maskf, r1aug, (((0,), (0,)), ((), ())),
                               preferred_element_type=jnp.float32)

    @pl.when(i == 0)
    def _():
        s4_o[...] = part

    @pl.when(i != 0)
    def _():
        s4_o[...] += part


def _tc_r2(k4r, x4pt2_768, r1, r1pt2):
    grid = N_PTS_PAD // PB
    return pl.pallas_call(
        _r2_body,
        grid=(grid,),
        in_specs=[pl.BlockSpec((1, PB, 1), lambda i: (i, 0, 0)),
                  pl.BlockSpec((768, 128), lambda i: (0, 0)),
                  pl.BlockSpec((PB, 256), lambda i: (i, 0)),
                  pl.BlockSpec((PB, 128), lambda i: (i, 0))],
        out_specs=[pl.BlockSpec((PB, 128), lambda i: (i, 0)),
                   pl.BlockSpec((768, 272), lambda i: (0, 0))],
        out_shape=[jax.ShapeDtypeStruct((N_PTS_PAD, 128), jnp.float32),
                   jax.ShapeDtypeStruct((768, 272), jnp.float32)],
    )(k4r, x4pt2_768, r1, r1pt2)


NW = 32
GNB = 4


def _gather_chunks(d):
    gch = 8192 // d
    per = N_PTS_PAD // NW
    return gch, per // gch


def _sc_gather(table, idx3):
    D = table.shape[1]
    GCH, GNCH = _gather_chunks(D)
    B = NW * GNCH * GCH
    mesh = plsc.VectorSubcoreMesh(core_axis_name="c", subcore_axis_name="s")

    @functools.partial(
        pl.kernel, mesh=mesh,
        out_type=jax.ShapeDtypeStruct((B, D), jnp.float32),
        scratch_types=(
            [pltpu.VMEM((GNCH, GCH), jnp.int32)]
            + [pltpu.VMEM((GCH, D), jnp.float32) for _ in range(GNB)]
            + [pltpu.SemaphoreType.DMA for _ in range(2 * GNB)]),
    )
    def k(table_hbm, idx_hbm, out_hbm, idx_v, *bs):
        bufs, gsems, ssems = bs[:GNB], bs[GNB:2 * GNB], bs[2 * GNB:]
        wid = lax.axis_index("s") * 2 + lax.axis_index("c")
        base = wid * (GNCH * GCH)
        pltpu.sync_copy(idx_hbm.at[wid], idx_v)
        gd = [None] * GNB
        sd = [None] * GNB
        for t in range(GNCH + GNB - 1):
            if t < GNCH:
                j = t % GNB
                if sd[j] is not None:
                    sd[j].wait()
                gd[j] = pltpu.async_copy(
                    table_hbm.at[idx_v.at[t]], bufs[j], gsems[j])
            if t >= GNB - 1:
                tt = t - (GNB - 1)
                jj = tt % GNB
                gd[jj].wait()
                sd[jj] = pltpu.async_copy(
                    bufs[jj], out_hbm.at[pl.ds(base + tt * GCH, GCH)],
                    ssems[jj])
        for j in range(GNB):
            if sd[j] is not None:
                sd[j].wait()

    return k(table, idx3)


def _seg_sum(x, idx, num):
    return jax.ops.segment_sum(x, idx, num_segments=num)


def _pad_rows(x, n):
    return jnp.pad(x, ((0, n - x.shape[0]),) + ((0, 0),) * (x.ndim - 1))


def kernel(voxel_coords, voxel_coords_batch, voxel_x, params):
    p = params
    c = voxel_coords.astype(jnp.int32)
    b = voxel_coords_batch.astype(jnp.int32)
    g = c // 16
    k4 = ((g[:, 0] * 7 + g[:, 1]) * 7 + g[:, 2]) * 2 + b
    k3 = k4 * 8 + ((c[:, 0] // 8) % 2) * 4 + ((c[:, 1] // 8) % 2) * 2 + ((c[:, 2] // 8) % 2)
    k2 = k3 * 8 + ((c[:, 0] // 4) % 2) * 4 + ((c[:, 1] // 4) % 2) * 2 + ((c[:, 2] // 4) % 2)
    k1 = k2 * 8 + ((c[:, 0] // 2) % 2) * 4 + ((c[:, 1] // 2) % 2) * 2 + ((c[:, 2] // 2) % 2)
    k0 = (c[:, 0] * 100 + c[:, 1]) * 100 + c[:, 2]
    k0 = k0 * 2 + b

    ones = jnp.ones((N_PTS, 1), jnp.float32)
    t0 = _seg_sum(jnp.concatenate([voxel_x, ones], 1), k0, N0D)
    t0p = t0[k0]
    cnt0 = jnp.maximum(t0p[:, 4:5], 1.0)
    f0p_w = jnp.concatenate(
        [t0p[:, :4] / cnt0, 1.0 / cnt0,
         jnp.zeros((N_PTS, 3), jnp.float32)], 1)
    f0p_w = _pad_rows(f0p_w, N_PTS_PAD)

    z0, zw40, r1, r1pt2 = _tc_pt0(f0p_w, p)

    t1 = _seg_sum(zw40[:N_PTS], k1, N1)
    t1 = _pad_rows(t1, N1P).reshape(N2P, 8, 40)

    x1, x2in = _tc_down(t1, p['s1a'], p['s1b'], N2P, 512, 32, 32, xw=128)
    x2in = x2in.reshape(N3P, 8, 33)
    x2, x3in = _tc_down(x2in, p['s2a'], p['s2b'], N3P, 64, 32, 64)
    x3in = x3in.reshape(N4P, 8, 65)
    x3, x4in = _tc_down(x3in, p['s3a'], p['s3b'], N4P, N4P, 64, 128)
    x4, x4pt2 = _tc_lvl4(x4in, p)

    k4r = jnp.pad(k4, (0, N_PTS_PAD - N_PTS), constant_values=N4P) \
             .reshape(N_PTS_PAD // PB, PB, 1)
    x4pt2_768 = _pad_rows(x4pt2, 768)
    r2p, s4f = _tc_r2(k4r, x4pt2_768, r1, r1pt2)
    s4 = s4f[:N4P]

    s2u = _seg_sum(jnp.concatenate([r2p[:N_PTS], ones], 1), k2, N2)
    s2u = jnp.pad(s2u, ((0, N2P - N2), (0, 136 - 129)))

    y2a = _tc_up4(s4, x4, x3, p)
    y2 = _tc_up3(y2a.reshape(N3P, 128), x2.reshape(N3P, 8, 64), p)
    y2 = y2.reshape(N2P, 128)
    yf = _tc_y3(y2, s2u, p)

    g2ch, g2n = _gather_chunks(256)
    g1ch, g1n = _gather_chunks(128)
    k2pad = jnp.pad(k2, (0, N_PTS_PAD - N_PTS)).reshape(NW, g2n, g2ch)
    k1pad = jnp.pad(k1, (0, N_PTS_PAD - N_PTS)).reshape(NW, g1n, g1ch)
    gyf = _sc_gather(yf, k2pad)
    gx1 = _sc_gather(x1.reshape(N1P, 128), k1pad)

    out = _tc_final(gyf, gx1, z0, r2p, p)
    return out[:N_PTS]


# speedup vs baseline: 4.8657x; 1.0768x over previous
import functools

import jax
import jax.numpy as jnp
from jax import lax
from jax.experimental import pallas as pl
from jax.experimental.pallas import tpu as pltpu
from jax.experimental.pallas import tpu_sc as plsc

N4, N3, N2, N1 = 686, 5488, 43904, 351232
N4P, N3P, N2P, N1P = 688, 5504, 44032, 352256
N0D = N1 * 8
N_PTS = 100_000
PB = 1024
N_PTS_PAD = 100_352

def _relu(x):
    return jnp.maximum(x, 0.0)


def _dot(a, w):
    return jax.lax.dot_general(a, w, (((1,), (0,)), ((), ())),
                               preferred_element_type=jnp.float32)


def _pt0_body(f0p, s1, s2, t1, t2, z0_o, zw40_o, r1_o, r1pt2_o):
    x = _relu(_dot(f0p[:, :4], s1[...]))
    z0 = _relu(_dot(x, s2[...]))
    w = f0p[:, 4:5]
    z0_o[...] = z0
    zw40_o[...] = jnp.concatenate(
        [z0 * w, w, jnp.zeros((z0.shape[0], 7), jnp.float32)], 1)
    r1 = _relu(_dot(z0, t1[...]))
    r1_o[...] = r1
    r1pt2_o[...] = _dot(r1, t2[...])


def _tc_pt0(f0p_w, p):
    grid = N_PTS_PAD // PB
    return pl.pallas_call(
        lambda f, s1, s2, t1, t2, a, b, c, d: _pt0_body(
            f[...], s1, s2, t1, t2, a, b, c, d),
        grid=(grid,),
        in_specs=[
            pl.BlockSpec((PB, 8), lambda i: (i, 0)),
            pl.BlockSpec((4, 32), lambda i: (0, 0)),
            pl.BlockSpec((32, 32), lambda i: (0, 0)),
            pl.BlockSpec((32, 256), lambda i: (0, 0)),
            pl.BlockSpec((256, 128), lambda i: (0, 0)),
        ],
        out_specs=[
            pl.BlockSpec((PB, 32), lambda i: (i, 0)),
            pl.BlockSpec((PB, 40), lambda i: (i, 0)),
            pl.BlockSpec((PB, 256), lambda i: (i, 0)),
            pl.BlockSpec((PB, 128), lambda i: (i, 0)),
        ],
        out_shape=[
            jax.ShapeDtypeStruct((N_PTS_PAD, 32), jnp.float32),
            jax.ShapeDtypeStruct((N_PTS_PAD, 40), jnp.float32),
            jax.ShapeDtypeStruct((N_PTS_PAD, 256), jnp.float32),
            jax.ShapeDtypeStruct((N_PTS_PAD, 128), jnp.float32),
        ],
    )(f0p_w, p['stem1'], p['stem2'], p['pt1'], p['pt2'])


def _down_body(tin, wa, wb, x_o, nxt_o, cin, cout, xw):
    R = tin.shape[0]
    s = jnp.zeros((R, cout), jnp.float32)
    m = jnp.zeros((R, 1), jnp.float32)
    for j in range(8):
        tj = tin[:, j, :]
        mj = tj[:, cin:cin + 1]
        aj = tj[:, :cin] / jnp.maximum(mj, 1.0)
        xj = _relu(_dot(_relu(_dot(aj, wa[...])), wb[...]))
        if xw > cout:
            xj = jnp.concatenate(
                [xj, jnp.zeros((R, xw - cout), jnp.float32)], 1)
        x_o[:, j, :] = xj
        s = s + xj[:, :cout]
        m = m + (mj > 0.5).astype(jnp.float32)
    nxt_o[:, :cout] = s
    nxt_o[:, cout:cout + 1] = m


def _tc_down(tbl, wa, wb, nrows_p, rblk, cin, cout, xw=None):
    if xw is None:
        xw = cout
    grid = nrows_p // rblk
    fin, fout = wa.shape[0], wa.shape[1]
    tw = tbl.shape[2]
    return pl.pallas_call(
        lambda t, a, b, xo, no: _down_body(t[...], a, b, xo, no, cin, cout, xw),
        grid=(grid,),
        in_specs=[
            pl.BlockSpec((rblk, 8, tw), lambda i: (i, 0, 0)),
            pl.BlockSpec((fin, fout), lambda i: (0, 0)),
            pl.BlockSpec((fout, cout), lambda i: (0, 0)),
        ],
        out_specs=[
            pl.BlockSpec((rblk, 8, xw), lambda i: (i, 0, 0)),
            pl.BlockSpec((rblk, cout + 1), lambda i: (i, 0)),
        ],
        out_shape=[
            jax.ShapeDtypeStruct((nrows_p, 8, xw), jnp.float32),
            jax.ShapeDtypeStruct((nrows_p, cout + 1), jnp.float32),
        ],
    )(tbl, wa, wb)


def _lvl4_body(x4in, s4a, s4b, t2, x4_o, x4pt2_o):
    a = x4in[:, :128] / jnp.maximum(x4in[:, 128:129], 1.0)
    x4 = _relu(_dot(_relu(_dot(a, s4a[...])), s4b[...]))
    x4_o[...] = x4
    x4pt2_o[...] = _dot(x4, t2[...])


def _tc_lvl4(x4in, p):
    return pl.pallas_call(
        _lvl4_body,
        in_specs=[pl.BlockSpec((N4P, 129), lambda: (0, 0)),
                  pl.BlockSpec((128, 256), lambda: (0, 0)),
                  pl.BlockSpec((256, 256), lambda: (0, 0)),
                  pl.BlockSpec((256, 128), lambda: (0, 0))],
        out_specs=[pl.BlockSpec((N4P, 256), lambda: (0, 0)),
                   pl.BlockSpec((N4P, 128), lambda: (0, 0))],
        out_shape=[jax.ShapeDtypeStruct((N4P, 256), jnp.float32),
                   jax.ShapeDtypeStruct((N4P, 128), jnp.float32)],
    )(x4in, p['s4a'], p['s4b'], p['pt2'])


def _up4_body(s4, x4, x3, u1a, u1bt, u1bb, u2a, y2a_o):
    y1in = x4[...] + s4[:, :256] / jnp.maximum(s4[:, 256:257], 1.0)
    y1v = _relu(_dot(y1in, u1a[...]))
    for j in range(8):
        y1j = _relu(_dot(y1v, u1bt[...]) + _dot(x3[:, j, :], u1bb[...]))
        y2a_o[:, j, :] = _relu(_dot(y1j, u2a[...]))


def _tc_up4(s4, x4, x3, p):
    return pl.pallas_call(
        _up4_body,
        in_specs=[pl.BlockSpec((N4P, 272), lambda: (0, 0)),
                  pl.BlockSpec((N4P, 256), lambda: (0, 0)),
                  pl.BlockSpec((N4P, 8, 128), lambda: (0, 0, 0)),
                  pl.BlockSpec((256, 256), lambda: (0, 0)),
                  pl.BlockSpec((256, 256), lambda: (0, 0)),
                  pl.BlockSpec((128, 256), lambda: (0, 0)),
                  pl.BlockSpec((256, 128), lambda: (0, 0))],
        out_specs=pl.BlockSpec((N4P, 8, 128), lambda: (0, 0, 0)),
        out_shape=jax.ShapeDtypeStruct((N4P, 8, 128), jnp.float32),
    )(s4, x4, x3, p['u1a'], p['u1b'][:256], p['u1b'][256:], p['u2a'])


def _up3_body(y2a, x2, u2bt, u2bb, y2_o):
    t = _dot(y2a[...], u2bt[...])
    for j in range(8):
        y2_o[:, j, :] = _relu(t + _dot(x2[:, j, :], u2bb[...]))


def _tc_up3(y2a, x2, p, rblk=64):
    grid = N3P // rblk
    return pl.pallas_call(
        _up3_body,
        grid=(grid,),
        in_specs=[pl.BlockSpec((rblk, 128), lambda i: (i, 0)),
                  pl.BlockSpec((rblk, 8, 64), lambda i: (i, 0, 0)),
                  pl.BlockSpec((128, 128), lambda i: (0, 0)),
                  pl.BlockSpec((64, 128), lambda i: (0, 0))],
        out_specs=pl.BlockSpec((rblk, 8, 128), lambda i: (i, 0, 0)),
        out_shape=jax.ShapeDtypeStruct((N3P, 8, 128), jnp.float32),
    )(y2a, x2, p['u2b'][:128], p['u2b'][128:])


def _y3_body(y2, s2u, u3a, yf_o):
    y3v = y2[...] + s2u[:, :128] / jnp.maximum(s2u[:, 128:129], 1.0)
    yf_o[:, :128] = y2[...]
    yf_o[:, 128:224] = _relu(_dot(y3v, u3a[...]))
    yf_o[:, 224:] = jnp.zeros((y2.shape[0], 32), jnp.float32)


def _tc_y3(y2, s2u, p, rblk=512):
    grid = N2P // rblk
    return pl.pallas_call(
        _y3_body,
        grid=(grid,),
        in_specs=[pl.BlockSpec((rblk, 128), lambda i: (i, 0)),
                  pl.BlockSpec((rblk, 136), lambda i: (i, 0)),
                  pl.BlockSpec((128, 96), lambda i: (0, 0))],
        out_specs=pl.BlockSpec((rblk, 256), lambda i: (i, 0)),
        out_shape=jax.ShapeDtypeStruct((N2P, 256), jnp.float32),
    )(y2, s2u, p['u3a'])


def _final_body(gyf, gx1, z0, r2p, u3bt, u3bb, u4a, u4bt, u4bb,
                pt3, clsw, clsb, out_o):
    gy3a = gyf[:, 128:224]
    z2 = gyf[:, :128] + r2p[...]
    y3p = _relu(_dot(gy3a, u3bt[...]) + _dot(gx1[:, :32], u3bb[...]))
    y4a = _relu(_dot(y3p, u4a[...]))
    y4p = _relu(_dot(y4a, u4bt[...]) + _dot(z0[...], u4bb[...]))
    z3 = y4p + _relu(_dot(z2, pt3[...]))
    out_o[...] = _dot(z3, clsw[...]) + clsb[...]


def _tc_final(gyf, gx1, z0, r2p, p):
    grid = N_PTS_PAD // PB
    return pl.pallas_call(
        _final_body,
        grid=(grid,),
        in_specs=[pl.BlockSpec((PB, 256), lambda i: (i, 0)),
                  pl.BlockSpec((PB, 128), lambda i: (i, 0)),
                  pl.BlockSpec((PB, 32), lambda i: (i, 0)),
                  pl.BlockSpec((PB, 128), lambda i: (i, 0)),
                  pl.BlockSpec((96, 96), lambda i: (0, 0)),
                  pl.BlockSpec((32, 96), lambda i: (0, 0)),
                  pl.BlockSpec((96, 96), lambda i: (0, 0)),
                  pl.BlockSpec((96, 96), lambda i: (0, 0)),
                  pl.BlockSpec((32, 96), lambda i: (0, 0)),
                  pl.BlockSpec((128, 96), lambda i: (0, 0)),
                  pl.BlockSpec((96, 20), lambda i: (0, 0)),
                  pl.BlockSpec((1, 20), lambda i: (0, 0))],
        out_specs=pl.BlockSpec((PB, 20), lambda i: (i, 0)),
        out_shape=jax.ShapeDtypeStruct((N_PTS_PAD, 20), jnp.float32),
    )(gyf, gx1, z0, r2p, p['u3b'][:96], p['u3b'][96:], p['u4a'],
      p['u4b'][:96], p['u4b'][96:], p['pt3'], p['cls_w'],
      p['cls_b'].reshape(1, 20))


def _r2_body(k4r, x4pt2, r1, r1pt2, r2_o, s4_o):
    i = pl.program_id(0)
    kb = k4r[0]
    mask = (kb == jax.lax.broadcasted_iota(jnp.int32, (PB, 768), 1))
    maskf = mask.astype(jnp.float32)
    g4 = _dot(maskf, x4pt2[...])
    r2_o[...] = _relu(g4 + r1pt2[...])
    r1aug = jnp.concatenate(
        [r1[...], jnp.ones((PB, 1), jnp.float32),
         jnp.zeros((PB, 15), jnp.float32)], 1)
    part = jax.lax.dot_general(maskf, r1aug, (((0,), (0,)), ((), ())),
                               preferred_element_type=jnp.float32)

    @pl.when(i == 0)
    def _():
        s4_o[...] = part

    @pl.when(i != 0)
    def _():
        s4_o[...] += part


def _tc_r2(k4r, x4pt2_768, r1, r1pt2):
    grid = N_PTS_PAD // PB
    return pl.pallas_call(
        _r2_body,
        grid=(grid,),
        in_specs=[pl.BlockSpec((1, PB, 1), lambda i: (i, 0, 0)),
                  pl.BlockSpec((768, 128), lambda i: (0, 0)),
                  pl.BlockSpec((PB, 256), lambda i: (i, 0)),
                  pl.BlockSpec((PB, 128), lambda i: (i, 0))],
        out_specs=[pl.BlockSpec((PB, 128), lambda i: (i, 0)),
                   pl.BlockSpec((768, 272), lambda i: (0, 0))],
        out_shape=[jax.ShapeDtypeStruct((N_PTS_PAD, 128), jnp.float32),
                   jax.ShapeDtypeStruct((768, 272), jnp.float32)],
    )(k4r, x4pt2_768, r1, r1pt2)


NW = 32
GNB = 4


def _gather_chunks(d):
    gch = 8192 // d
    per = N_PTS_PAD // NW
    return gch, per // gch


def _sc_gather(table, idx3):
    D = table.shape[1]
    GCH, GNCH = _gather_chunks(D)
    B = NW * GNCH * GCH
    mesh = plsc.VectorSubcoreMesh(core_axis_name="c", subcore_axis_name="s")

    @functools.partial(
        pl.kernel, mesh=mesh,
        out_type=jax.ShapeDtypeStruct((B, D), jnp.float32),
        scratch_types=(
            [pltpu.VMEM((GNCH, GCH), jnp.int32)]
            + [pltpu.VMEM((GCH, D), jnp.float32) for _ in range(GNB)]
            + [pltpu.SemaphoreType.DMA for _ in range(2 * GNB)]),
    )
    def k(table_hbm, idx_hbm, out_hbm, idx_v, *bs):
        bufs, gsems, ssems = bs[:GNB], bs[GNB:2 * GNB], bs[2 * GNB:]
        wid = lax.axis_index("s") * 2 + lax.axis_index("c")
        base = wid * (GNCH * GCH)
        pltpu.sync_copy(idx_hbm.at[wid], idx_v)
        gd = [None] * GNB
        sd = [None] * GNB
        for t in range(GNCH + GNB - 1):
            if t < GNCH:
                j = t % GNB
                if sd[j] is not None:
                    sd[j].wait()
                gd[j] = pltpu.async_copy(
                    table_hbm.at[idx_v.at[t]], bufs[j], gsems[j])
            if t >= GNB - 1:
                tt = t - (GNB - 1)
                jj = tt % GNB
                gd[jj].wait()
                sd[jj] = pltpu.async_copy(
                    bufs[jj], out_hbm.at[pl.ds(base + tt * GCH, GCH)],
                    ssems[jj])
        for j in range(GNB):
            if sd[j] is not None:
                sd[j].wait()

    return k(table, idx3)


def _seg_sum(x, idx, num):
    return jax.ops.segment_sum(x, idx, num_segments=num,
                               indices_are_sorted=True)


def _pad_rows(x, n):
    return jnp.pad(x, ((0, n - x.shape[0]),) + ((0, 0),) * (x.ndim - 1))


def kernel(voxel_coords, voxel_coords_batch, voxel_x, params):
    p = params
    c = voxel_coords.astype(jnp.int32)
    b = voxel_coords_batch.astype(jnp.int32)
    g = c // 16
    k4 = ((g[:, 0] * 7 + g[:, 1]) * 7 + g[:, 2]) * 2 + b
    k3 = k4 * 8 + ((c[:, 0] // 8) % 2) * 4 + ((c[:, 1] // 8) % 2) * 2 + ((c[:, 2] // 8) % 2)
    k2 = k3 * 8 + ((c[:, 0] // 4) % 2) * 4 + ((c[:, 1] // 4) % 2) * 2 + ((c[:, 2] // 4) % 2)
    k1 = k2 * 8 + ((c[:, 0] // 2) % 2) * 4 + ((c[:, 1] // 2) % 2) * 2 + ((c[:, 2] // 2) % 2)
    k0 = k1 * 8 + (c[:, 0] % 2) * 4 + (c[:, 1] % 2) * 2 + (c[:, 2] % 2)

    k0, perm = lax.sort_key_val(k0, lax.iota(jnp.int32, N_PTS))
    voxel_x = voxel_x[perm]
    k1 = k0 // 8
    k2 = k0 // 64
    k4 = k0 // 4096

    seg0 = jnp.cumsum(
        jnp.concatenate([jnp.zeros((1,), jnp.int32),
                         (k0[1:] != k0[:-1]).astype(jnp.int32)]))
    ones = jnp.ones((N_PTS, 1), jnp.float32)
    t0 = _seg_sum(jnp.concatenate([voxel_x, ones], 1), seg0, N_PTS)
    t0p = t0[seg0]
    cnt0 = jnp.maximum(t0p[:, 4:5], 1.0)
    f0p_w = jnp.concatenate(
        [t0p[:, :4] / cnt0, 1.0 / cnt0,
         jnp.zeros((N_PTS, 3), jnp.float32)], 1)
    f0p_w = _pad_rows(f0p_w, N_PTS_PAD)

    z0, zw40, r1, r1pt2 = _tc_pt0(f0p_w, p)

    t1 = _seg_sum(zw40[:N_PTS], k1, N1)
    t1 = _pad_rows(t1, N1P).reshape(N2P, 8, 40)

    x1, x2in = _tc_down(t1, p['s1a'], p['s1b'], N2P, 512, 32, 32, xw=128)
    x2in = x2in.reshape(N3P, 8, 33)
    x2, x3in = _tc_down(x2in, p['s2a'], p['s2b'], N3P, 64, 32, 64)
    x3in = x3in.reshape(N4P, 8, 65)
    x3, x4in = _tc_down(x3in, p['s3a'], p['s3b'], N4P, N4P, 64, 128)
    x4, x4pt2 = _tc_lvl4(x4in, p)

    k4r = jnp.pad(k4, (0, N_PTS_PAD - N_PTS), constant_values=N4P) \
             .reshape(N_PTS_PAD // PB, PB, 1)
    x4pt2_768 = _pad_rows(x4pt2, 768)
    r2p, s4f = _tc_r2(k4r, x4pt2_768, r1, r1pt2)
    s4 = s4f[:N4P]

    s2u = _seg_sum(jnp.concatenate([r2p[:N_PTS], ones], 1), k2, N2)
    s2u = jnp.pad(s2u, ((0, N2P - N2), (0, 136 - 129)))

    y2a = _tc_up4(s4, x4, x3, p)
    y2 = _tc_up3(y2a.reshape(N3P, 128), x2.reshape(N3P, 8, 64), p)
    y2 = y2.reshape(N2P, 128)
    yf = _tc_y3(y2, s2u, p)

    g2ch, g2n = _gather_chunks(256)
    g1ch, g1n = _gather_chunks(128)
    k2pad = jnp.pad(k2, (0, N_PTS_PAD - N_PTS)).reshape(NW, g2n, g2ch)
    k1pad = jnp.pad(k1, (0, N_PTS_PAD - N_PTS)).reshape(NW, g1n, g1ch)
    gyf = _sc_gather(yf, k2pad)
    gx1 = _sc_gather(x1.reshape(N1P, 128), k1pad)

    out = _tc_final(gyf, gx1, z0, r2p, p)
    return jnp.zeros_like(out[:N_PTS]).at[perm].set(out[:N_PTS])
